# Initial kernel scaffold; baseline (speedup 1.0000x reference)
import jax
import jax.numpy as jnp
from jax.experimental import pallas as pl


def kernel(x, edge_index, w1, b1, w2, b2, gate_w, gate_b, w_ih, w_hh, b_ih, b_hh, d1_w, d1_b, d2_w, d2_b):
    raise NotImplementedError("write your pallas kernel here")


# speedup vs baseline: 1.0002x; 1.0002x over previous
import jax
import jax.numpy as jnp
from jax.experimental import pallas as pl
from jax.experimental.pallas import tpu as pltpu

N = 10000
H1 = 32
H2 = 128
T = 8


def _silu(v):
    return v * jax.nn.sigmoid(v)


def _head_kernel(pooled_ref, w_ih_ref, w_hh_ref, b_ih_ref, b_hh_ref,
                 d1_w_ref, d1_b_ref, d2_w_ref, d2_b_ref, out_ref):
    pooled = pooled_ref[...]
    w_ih = w_ih_ref[...]
    w_hh = w_hh_ref[...]
    b_ih = b_ih_ref[...]
    b_hh = b_hh_ref[...]
    h = jnp.zeros((1, H2), jnp.float32)
    for t in range(T):
        xt = pooled[t][None, :]
        gi = jnp.dot(xt, w_ih.T, preferred_element_type=jnp.float32) + b_ih
        gh = jnp.dot(h, w_hh.T, preferred_element_type=jnp.float32) + b_hh
        r = jax.nn.sigmoid(gi[:, :H2] + gh[:, :H2])
        z = jax.nn.sigmoid(gi[:, H2:2 * H2] + gh[:, H2:2 * H2])
        n = jnp.tanh(gi[:, 2 * H2:] + r * gh[:, 2 * H2:])
        h = (1.0 - z) * n + z * h
    o = _silu(jnp.dot(h, d1_w_ref[...], preferred_element_type=jnp.float32) + d1_b_ref[...])
    o = _silu(jnp.dot(o, d2_w_ref[...], preferred_element_type=jnp.float32) + d2_b_ref[...])
    out_ref[...] = jax.nn.sigmoid(o[:, :4])


def kernel(x, edge_index, w1, b1, w2, b2, gate_w, gate_b, w_ih, w_hh, b_ih, b_hh, d1_w, d1_b, d2_w, d2_b):
    src = edge_index[0]
    dst = edge_index[1]
    ones = jnp.ones((src.shape[0],), jnp.float32)
    deg_out = jax.ops.segment_sum(ones, src, num_segments=N)
    deg_in = jax.ops.segment_sum(ones, dst, num_segments=N)
    norm_out = jnp.maximum(deg_out, 1.0) ** -0.5
    norm_in = jnp.maximum(deg_in, 1.0) ** -0.5

    def gconv(h, w, b):
        h = h * norm_out[:, None]
        msg = h[src]
        agg = jax.ops.segment_sum(msg, dst, num_segments=N)
        agg = agg * norm_in[:, None]
        return _silu(agg @ w + b)

    def per_frame(xf):
        h = gconv(xf, w1, b1)
        h = gconv(h, w2, b2)
        gate = h @ gate_w + gate_b
        a = jax.nn.softmax(gate, axis=0)
        return jnp.sum(a * h, axis=0)

    pooled = jax.vmap(per_frame)(x)

    out = pl.pallas_call(
        _head_kernel,
        out_shape=jax.ShapeDtypeStruct((1, 4), jnp.float32),
    )(pooled, w_ih, w_hh, b_ih.reshape(1, -1), b_hh.reshape(1, -1),
      d1_w, d1_b.reshape(1, -1), d2_w, d2_b.reshape(1, -1))
    return out


# speedup vs baseline: 11.9721x; 11.9695x over previous
import functools

import jax
import jax.numpy as jnp
from jax import lax
from jax.experimental import pallas as pl
from jax.experimental.pallas import tpu as pltpu
from jax.experimental.pallas import tpu_sc as plsc

TT, NN, EE = 8, 10000, 160000
H1, H2 = 32, 128
NP = 10240
NSL = NP // 16
CH = 128
CPT = 79
EA = 16 * CPT * CH


def _vsmesh():
    return plsc.VectorSubcoreMesh(core_axis_name="c", subcore_axis_name="s",
                                  num_cores=2, num_subcores=16)


def _fill(ref, rows, vec):
    ncol = ref.shape[1] // 16

    def body(i, _):
        for k in range(ncol):
            ref[i, pl.ds(k * 16, 16)] = vec
        return 0
    lax.fori_loop(0, rows, body, 0)


def _zero_slice(zbuf, acc, sid):
    for k in range(NSL // CH):
        pltpu.sync_copy(zbuf, acc.at[pl.ds(sid * NSL + k * CH, CH)])


def _deg_body(src_hbm, dst_hbm, degs, idx_v, zeros_v, acc):
    cid = lax.axis_index("c")
    sid = lax.axis_index("s")

    @pl.when(cid == 0)
    def _():
        pltpu.sync_copy(src_hbm.at[sid], idx_v)

    @pl.when(cid == 1)
    def _():
        pltpu.sync_copy(dst_hbm.at[sid], idx_v)

    lane = lax.iota(jnp.int32, 16)
    one_row = jnp.where(lane == 0, 1.0, 0.0).astype(jnp.float32)
    _fill(zeros_v, CH, jnp.zeros((16,), jnp.float32))
    _zero_slice(zeros_v, acc, sid)
    plsc.subcore_barrier()
    _fill(zeros_v, CH, one_row)
    ones_v = zeros_v

    def chunk(j, _):
        pltpu.sync_copy(ones_v, acc.at[idx_v.at[j]], add=True)
        return 0
    lax.fori_loop(0, CPT, chunk, 0)
    plsc.subcore_barrier()
    pltpu.sync_copy(acc.at[pl.ds(sid * NSL, NSL)],
                    degs.at[cid, pl.ds(sid * NSL, NSL)])


def _deg_call(srcp, dstp):
    k = functools.partial(
        pl.kernel,
        out_type=jax.ShapeDtypeStruct((2, NP, 16), jnp.float32),
        mesh=_vsmesh(),
        compiler_params=pltpu.CompilerParams(needs_layout_passes=False),
        scratch_types=[
            pltpu.VMEM((CPT, CH), jnp.int32),
            pltpu.VMEM((CH, 16), jnp.float32),
            pltpu.VMEM_SHARED((NP, 16), jnp.float32),
        ],
    )(_deg_body)
    return k(srcp, dstp)


def _prep_body(degT_ref, x2T_ref, xnt_ref, no_ref, ni32_ref):
    degT = degT_ref[...]
    no = lax.rsqrt(jnp.maximum(degT[:, 0:1], 1.0))
    ni = lax.rsqrt(jnp.maximum(degT[:, 1:2], 1.0))
    x2T = x2T_ref[...]
    z124 = jnp.zeros((NP, 124), jnp.float32)
    xnt_ref[0] = jnp.concatenate([x2T[:, 0:4] * no, z124], axis=1)
    xnt_ref[1] = jnp.concatenate([x2T[:, 4:8] * no, z124], axis=1)
    no_ref[...] = no
    ni32_ref[...] = jnp.broadcast_to(ni, (NP, H1))


def _prep_call(degT, x2T):
    return pl.pallas_call(
        _prep_body,
        out_shape=[
            jax.ShapeDtypeStruct((2, NP, 128), jnp.float32),
            jax.ShapeDtypeStruct((NP, 1), jnp.float32),
            jax.ShapeDtypeStruct((NP, H1), jnp.float32),
        ],
    )(degT, x2T)


def _l1_body(src_hbm, dst_hbm, xnt_hbm, s1_out, src_v, dst_v, buf, acc):
    cid = lax.axis_index("c")
    sid = lax.axis_index("s")
    pltpu.sync_copy(src_hbm.at[sid], src_v)
    pltpu.sync_copy(dst_hbm.at[sid], dst_v)

    _fill(buf, CH, jnp.zeros((16,), jnp.float32))
    _zero_slice(buf, acc, sid)
    off = cid * NP

    def mkidx(j, _):
        for g in range(8):
            src_v[j, pl.ds(g * 16, 16)] = src_v[j, pl.ds(g * 16, 16)] + off
        return 0
    lax.fori_loop(0, CPT, mkidx, 0)
    plsc.subcore_barrier()

    def chunk(j, _):
        pltpu.sync_copy(xnt_hbm.at[src_v.at[j]], buf)
        pltpu.sync_copy(buf, acc.at[dst_v.at[j]], add=True)
        return 0
    lax.fori_loop(0, CPT, chunk, 0)
    plsc.subcore_barrier()
    pltpu.sync_copy(acc.at[pl.ds(sid * NSL, NSL)],
                    s1_out.at[cid, pl.ds(sid * NSL, NSL)])


def _l1_call(srcp, dstp, xnt2):
    k = functools.partial(
        pl.kernel,
        out_type=jax.ShapeDtypeStruct((2, NP, 128), jnp.float32),
        mesh=_vsmesh(),
        compiler_params=pltpu.CompilerParams(needs_layout_passes=False),
        scratch_types=[
            pltpu.VMEM((CPT, CH), jnp.int32),
            pltpu.VMEM((CPT, CH), jnp.int32),
            pltpu.VMEM((CH, 128), jnp.float32),
            pltpu.VMEM_SHARED((NP, 128), jnp.float32),
        ],
    )(_l1_body)
    return k(srcp, dstp, xnt2)


def _a1_body(s1p_ref, degT_ref, a1m_ref):
    ni = lax.rsqrt(jnp.maximum(degT_ref[...][:, 1:2], 1.0))
    a1m_ref[...] = jnp.concatenate([s1p_ref[0][:, 0:4],
                                    s1p_ref[1][:, 0:4]], axis=1) * ni


def _a1_call(s1p, degT):
    return pl.pallas_call(
        _a1_body,
        out_shape=jax.ShapeDtypeStruct((NP, 8), jnp.float32),
    )(s1p, degT)


def _m1_body(a1m_ref, no_ref, w1_ref, b1_ref, m1_ref):
    a1m = a1m_ref[...]
    no = no_ref[...]
    w1 = w1_ref[...]
    b1 = b1_ref[...]
    cols = []
    for f in range(TT):
        v = a1m[:, f:f + 1] * w1 + b1
        cols.append(_silu(v) * no)
    m1_ref[...] = jnp.concatenate(cols, axis=1)


def _m1_call(a1m, no_col, w1, b1):
    return pl.pallas_call(
        _m1_body,
        out_shape=jax.ShapeDtypeStruct((NP, TT * 128), jnp.float32),
    )(a1m, no_col, w1, b1)


def _l2_body(src_hbm, dst_hbm, m1_hbm, s2_out, src_v, dst_v, buf, acc):
    cid = lax.axis_index("c")
    sid = lax.axis_index("s")
    pltpu.sync_copy(src_hbm.at[sid], src_v)
    pltpu.sync_copy(dst_hbm.at[sid], dst_v)
    _fill(buf, CH, jnp.zeros((16,), jnp.float32))

    def shift(delta):
        def mkidx(j, _):
            for g in range(8):
                src_v[j, pl.ds(g * 16, 16)] = src_v[j, pl.ds(g * 16, 16)] + delta
            return 0
        lax.fori_loop(0, CPT, mkidx, 0)
    shift(cid * 4 * NP)

    def frame(f, _):
        _zero_slice(buf, acc, sid)
        fg = cid * 4 + f
        plsc.subcore_barrier()

        def chunk(j, _):
            pltpu.sync_copy(m1_hbm.at[src_v.at[j]], buf)
            pltpu.sync_copy(buf, acc.at[dst_v.at[j]], add=True)
            return 0
        lax.fori_loop(0, CPT, chunk, 0)
        plsc.subcore_barrier()
        pltpu.sync_copy(acc.at[pl.ds(sid * NSL, NSL)],
                        s2_out.at[fg, pl.ds(sid * NSL, NSL)])
        shift(NP)
        return 0
    lax.fori_loop(0, 4, frame, 0)


def _l2_call(srcp, dstp, m1flat):
    k = functools.partial(
        pl.kernel,
        out_type=jax.ShapeDtypeStruct((TT, NP, 128), jnp.float32),
        mesh=_vsmesh(),
        compiler_params=pltpu.CompilerParams(needs_layout_passes=False),
        scratch_types=[
            pltpu.VMEM((CPT, CH), jnp.int32),
            pltpu.VMEM((CPT, CH), jnp.int32),
            pltpu.VMEM((CH, 128), jnp.float32),
            pltpu.VMEM_SHARED((NP, 128), jnp.float32),
        ],
    )(_l2_body)
    return k(srcp, dstp, m1flat)


def _silu(v):
    return v * jax.nn.sigmoid(v)


NBLK = 10
BLK = NP // NBLK


def _pool_body(s2_ref, w2_ref, b2_ref, gw_ref, gb_ref, out_ref,
               acc_ref, m_ref, s_ref):
    f = pl.program_id(0)
    j = pl.program_id(1)

    @pl.when(j == 0)
    def _():
        acc_ref[...] = jnp.zeros_like(acc_ref)
        m_ref[0, 0] = -1e30
        s_ref[0, 0] = 0.0

    a2 = s2_ref[0]
    h2 = _silu(jnp.dot(a2, w2_ref[...],
                       preferred_element_type=jnp.float32) + b2_ref[...])
    g = jnp.sum(h2 * gw_ref[...], axis=1, keepdims=True) + gb_ref[0, 0]
    rows = j * BLK + lax.broadcasted_iota(jnp.int32, (BLK, 1), 0)
    g = jnp.where(rows < NN, g, -1e30)
    bm = jnp.max(g)
    m_old = m_ref[0, 0]
    m_new = jnp.maximum(m_old, bm)
    scale = jnp.exp(m_old - m_new)
    p = jnp.exp(g - m_new)
    m_ref[0, 0] = m_new
    s_ref[0, 0] = s_ref[0, 0] * scale + jnp.sum(p)
    acc_ref[...] = acc_ref[...] * scale + jnp.sum(p * h2, axis=0,
                                                  keepdims=True)

    @pl.when(j == NBLK - 1)
    def _():
        out_ref[pl.ds(f, 1), :] = acc_ref[...] / s_ref[0, 0]


def _pool_call(s2, w2, b2, gw, gb):
    return pl.pallas_call(
        _pool_body,
        grid=(TT, NBLK),
        in_specs=[
            pl.BlockSpec((1, BLK, H1), lambda f, j: (f, j, 0)),
            pl.BlockSpec((H1, H2), lambda f, j: (0, 0)),
            pl.BlockSpec((1, H2), lambda f, j: (0, 0)),
            pl.BlockSpec((1, H2), lambda f, j: (0, 0)),
            pl.BlockSpec((1, 1), lambda f, j: (0, 0),
                         memory_space=pltpu.SMEM),
        ],
        out_specs=pl.BlockSpec((TT, H2), lambda f, j: (0, 0)),
        out_shape=jax.ShapeDtypeStruct((TT, H2), jnp.float32),
        scratch_shapes=[
            pltpu.VMEM((1, H2), jnp.float32),
            pltpu.SMEM((1, 1), jnp.float32),
            pltpu.SMEM((1, 1), jnp.float32),
        ],
    )(s2, w2, b2, gw, gb)


def _head_body(pooled_ref, w_ih_ref, w_hh_ref, b_ih_ref, b_hh_ref,
               d1_w_ref, d1_b_ref, d2_w_ref, d2_b_ref, out_ref):
    pooled = pooled_ref[...]
    w_ih = w_ih_ref[...]
    w_hh = w_hh_ref[...]
    b_ih = b_ih_ref[...]
    b_hh = b_hh_ref[...]
    h = jnp.zeros((1, H2), jnp.float32)
    for t in range(TT):
        xt = pooled[t][None, :]
        gi = jnp.dot(xt, w_ih.T, preferred_element_type=jnp.float32) + b_ih
        gh = jnp.dot(h, w_hh.T, preferred_element_type=jnp.float32) + b_hh
        r = jax.nn.sigmoid(gi[:, :H2] + gh[:, :H2])
        z = jax.nn.sigmoid(gi[:, H2:2 * H2] + gh[:, H2:2 * H2])
        n = jnp.tanh(gi[:, 2 * H2:] + r * gh[:, 2 * H2:])
        h = (1.0 - z) * n + z * h
    o = _silu(jnp.dot(h, d1_w_ref[...],
                      preferred_element_type=jnp.float32) + d1_b_ref[...])
    o = _silu(jnp.dot(o, d2_w_ref[...],
                      preferred_element_type=jnp.float32) + d2_b_ref[...])
    out_ref[...] = jax.nn.sigmoid(o[:, :4])


def _head_call(pooled, w_ih, w_hh, b_ih, b_hh, d1_w, d1_b, d2_w, d2_b):
    return pl.pallas_call(
        _head_body,
        out_shape=jax.ShapeDtypeStruct((1, 4), jnp.float32),
    )(pooled, w_ih, w_hh, b_ih.reshape(1, -1), b_hh.reshape(1, -1),
      d1_w, d1_b.reshape(1, -1), d2_w, d2_b.reshape(1, -1))


def kernel(x, edge_index, w1, b1, w2, b2, gate_w, gate_b, w_ih, w_hh,
           b_ih, b_hh, d1_w, d1_b, d2_w, d2_b):
    src = edge_index[0]
    dst = edge_index[1]
    pad = jnp.full((EA - EE,), NP - 1, jnp.int32)
    srcp = jnp.concatenate([src, pad]).reshape(16, CPT, CH)
    dstp = jnp.concatenate([dst, pad]).reshape(16, CPT, CH)
    x2 = jnp.pad(x[..., 0], ((0, 0), (0, NP - NN)))

    degs = _deg_call(srcp, dstp)
    degT = degs[:, :, 0].T
    xnt, no_col, ni32 = _prep_call(degT, x2.T)
    xnt2 = xnt.reshape(2 * NP, 128)

    s1p = _l1_call(srcp, dstp, xnt2)
    a1m = _a1_call(s1p, degT)
    w1p = jnp.pad(w1, ((0, 0), (0, 128 - H1)))
    b1p = jnp.pad(b1.reshape(1, -1), ((0, 0), (0, 128 - H1)))
    m1w = _m1_call(a1m, no_col, w1p, b1p)
    m1flat = m1w.reshape(NP, TT, 128).transpose(1, 0, 2).reshape(TT * NP, 128)

    s2 = _l2_call(srcp, dstp, m1flat)
    a2 = s2[:, :, 0:H1] * ni32[None]
    pooled = _pool_call(a2, w2, b2.reshape(1, -1),
                        gate_w.reshape(1, -1), gate_b.reshape(1, 1))
    return _head_call(pooled, w_ih, w_hh, b_ih, b_hh, d1_w, d1_b, d2_w, d2_b)
